# baseline (speedup 1.0000x reference)
import functools

import jax
import jax.numpy as jnp
from jax import lax
from jax.experimental import pallas as pl
from jax.experimental.pallas import tpu as pltpu
from jax.experimental.pallas import tpu_sc as plsc

N = 50000
NPAD = 50176
IN_DIM = 24
HID = 256
OUT_DIM = 12
E = 800000
EPAD = 819200
NC = 2
NS = 16
CW = 16
NCH = HID // CW
EPT = EPAD // NS
BB = 256
KK = 10
NR = 3
NSUPER = EPT // (BB * KK)
STRIPE = NPAD // NS


def _dot(a, b):
    return lax.dot_general(a, b, (((1,), (0,)), ((), ())),
                           preferred_element_type=jnp.float32)


_BR = 1024
_GRID = NPAD // _BR


def _enc_body(x_ref, w1_ref, b1_ref, w2_ref, b2_ref, o_ref):
    h = jnp.maximum(_dot(x_ref[...], w1_ref[...]) + b1_ref[...], 0.0)
    o_ref[...] = jnp.maximum(_dot(h, w2_ref[...]) + b2_ref[...], 0.0)


def _encoder(xp, w1, b1, w2, b2):
    return pl.pallas_call(
        _enc_body,
        grid=(_GRID,),
        in_specs=[
            pl.BlockSpec((_BR, IN_DIM), lambda i: (i, 0)),
            pl.BlockSpec((IN_DIM, HID // 2), lambda i: (0, 0)),
            pl.BlockSpec((1, HID // 2), lambda i: (0, 0)),
            pl.BlockSpec((HID // 2, HID), lambda i: (0, 0)),
            pl.BlockSpec((1, HID), lambda i: (0, 0)),
        ],
        out_specs=pl.BlockSpec((_BR, HID), lambda i: (i, 0)),
        out_shape=jax.ShapeDtypeStruct((NPAD, HID), jnp.float32),
    )(xp, w1, b1.reshape(1, -1), w2, b2.reshape(1, -1))


def _combine_body(agg_ref, cnt_ref, h_ref, wl_ref, bl_ref, wr_ref, o_ref):
    inv = 1.0 / jnp.maximum(cnt_ref[...], 1.0)
    mean = agg_ref[...] * inv
    o_ref[...] = jnp.maximum(
        _dot(mean, wl_ref[...]) + bl_ref[...] + _dot(h_ref[...], wr_ref[...]),
        0.0)


def _combine(agg, cnt, h, wl, bl, wr):
    return pl.pallas_call(
        _combine_body,
        grid=(_GRID,),
        in_specs=[
            pl.BlockSpec((_BR, HID), lambda i: (i, 0)),
            pl.BlockSpec((_BR, 1), lambda i: (i, 0)),
            pl.BlockSpec((_BR, HID), lambda i: (i, 0)),
            pl.BlockSpec((HID, HID), lambda i: (0, 0)),
            pl.BlockSpec((1, HID), lambda i: (0, 0)),
            pl.BlockSpec((HID, HID), lambda i: (0, 0)),
        ],
        out_specs=pl.BlockSpec((_BR, HID), lambda i: (i, 0)),
        out_shape=jax.ShapeDtypeStruct((NPAD, HID), jnp.float32),
    )(agg, cnt, h, wl, bl.reshape(1, -1), wr)


def _combine_ro_body(agg_ref, cnt_ref, h_ref, wl_ref, bl_ref, wr_ref,
                     rw1_ref, rb1_ref, rw2_ref, rb2_ref, o_ref):
    inv = 1.0 / jnp.maximum(cnt_ref[...], 1.0)
    mean = agg_ref[...] * inv
    t = jnp.maximum(
        _dot(mean, wl_ref[...]) + bl_ref[...] + _dot(h_ref[...], wr_ref[...]),
        0.0)
    t = jnp.maximum(_dot(t, rw1_ref[...]) + rb1_ref[...], 0.0)
    o_ref[...] = _dot(t, rw2_ref[...]) + rb2_ref[...]


def _combine_readout(agg, cnt, h, wl, bl, wr, rw1, rb1, rw2, rb2):
    return pl.pallas_call(
        _combine_ro_body,
        grid=(_GRID,),
        in_specs=[
            pl.BlockSpec((_BR, HID), lambda i: (i, 0)),
            pl.BlockSpec((_BR, 1), lambda i: (i, 0)),
            pl.BlockSpec((_BR, HID), lambda i: (i, 0)),
            pl.BlockSpec((HID, HID), lambda i: (0, 0)),
            pl.BlockSpec((1, HID), lambda i: (0, 0)),
            pl.BlockSpec((HID, HID), lambda i: (0, 0)),
            pl.BlockSpec((HID, HID // 2), lambda i: (0, 0)),
            pl.BlockSpec((1, HID // 2), lambda i: (0, 0)),
            pl.BlockSpec((HID // 2, OUT_DIM), lambda i: (0, 0)),
            pl.BlockSpec((1, OUT_DIM), lambda i: (0, 0)),
        ],
        out_specs=pl.BlockSpec((_BR, OUT_DIM), lambda i: (i, 0)),
        out_shape=jax.ShapeDtypeStruct((NPAD, OUT_DIM), jnp.float32),
    )(agg, cnt, h, wl, bl.reshape(1, -1), wr,
      rw1, rb1.reshape(1, -1), rw2, rb2.reshape(1, -1))


@functools.cache
def _make_sc_kernel(with_count):
    mesh = plsc.VectorSubcoreMesh(core_axis_name="c", subcore_axis_name="s",
                                  num_cores=NC, num_subcores=NS)
    out_type = [jax.ShapeDtypeStruct((NPAD, HID), jnp.float32)]
    if with_count:
        out_type.append(jax.ShapeDtypeStruct((NPAD, CW), jnp.float32))

    scratch = [
        pltpu.VMEM((KK, BB), jnp.int32),
        pltpu.VMEM((KK, BB), jnp.int32),
    ] + [pltpu.VMEM((BB, CW), jnp.float32) for _ in range(NR)] + [
        pltpu.VMEM_SHARED((NPAD, CW), jnp.float32),
    ] + [pltpu.SemaphoreType.DMA for _ in range(2 * NR)]

    def body(h8, idx8, dstr, zeros_h, ones_h, *rest):
        if with_count:
            agg_out, cnt_out = rest[0], rest[1]
            rest = rest[2:]
        else:
            agg_out = rest[0]
            cnt_out = None
            rest = rest[1:]
        idx_all, dst_all = rest[0], rest[1]
        rows = rest[2:2 + NR]
        slab = rest[2 + NR]
        gsem = rest[3 + NR:3 + 2 * NR]
        ssem = rest[3 + 2 * NR:3 + 3 * NR]

        cid = lax.axis_index("c")
        sid = lax.axis_index("s")
        stripe0 = sid * STRIPE

        def zero_slab():
            pltpu.sync_copy(zeros_h, slab.at[pl.ds(stripe0, STRIPE), :])

        def gather_pass(ch):
            zero_slab()
            plsc.subcore_barrier()

            def superb(sb, carry):
                row0 = sid * (EPT // BB) + sb * KK
                pltpu.sync_copy(dstr.at[pl.ds(row0, KK), :], dst_all)
                pltpu.sync_copy(idx8.at[ch, pl.ds(row0, KK), :], idx_all)
                gd = [None] * NR
                sd = [None] * NR
                for k in range(KK):
                    buf = k % NR
                    if sd[buf] is not None:
                        sd[buf].wait()
                    gd[buf] = pltpu.async_copy(
                        h8.at[idx_all.at[k]], rows[buf], gsem[buf])
                    if k >= NR - 1:
                        j = k - (NR - 1)
                        jb = j % NR
                        gd[jb].wait()
                        sd[jb] = pltpu.async_copy(
                            rows[jb], slab.at[dst_all.at[j]],
                            ssem[jb], add=True)
                for j in range(KK - (NR - 1), KK):
                    jb = j % NR
                    gd[jb].wait()
                    sd[jb] = pltpu.async_copy(
                        rows[jb], slab.at[dst_all.at[j]],
                        ssem[jb], add=True)
                for d in sd:
                    if d is not None:
                        d.wait()
                return carry

            lax.fori_loop(0, NSUPER, superb, 0)
            plsc.subcore_barrier()
            pltpu.sync_copy(
                slab.at[pl.ds(stripe0, STRIPE), :],
                agg_out.at[pl.ds(stripe0, STRIPE), pl.ds(ch * CW, CW)])
            plsc.subcore_barrier()

        for j in range(NCH // NC):
            gather_pass(cid * (NCH // NC) + j)

        if with_count:
            @pl.when(cid == 0)
            def _count_pass():
                zero_slab()
                pltpu.sync_copy(ones_h, rows[0])
                plsc.subcore_barrier()

                def superb(sb, carry):
                    row0 = sid * (EPT // BB) + sb * KK
                    pltpu.sync_copy(dstr.at[pl.ds(row0, KK), :], dst_all)
                    sds = []
                    for k in range(KK):
                        sds.append(pltpu.async_copy(
                            rows[0], slab.at[dst_all.at[k]], ssem[0],
                            add=True))
                    for sd in sds:
                        sd.wait()
                    return carry

                lax.fori_loop(0, NSUPER, superb, 0)
                plsc.subcore_barrier()
                pltpu.sync_copy(slab.at[pl.ds(stripe0, STRIPE), :],
                                cnt_out.at[pl.ds(stripe0, STRIPE), :])

    return pl.kernel(
        body,
        out_type=tuple(out_type) if with_count else out_type[0],
        mesh=mesh,
        scratch_types=scratch,
        compiler_params=pltpu.CompilerParams(use_tc_tiling_on_sc=False),
    )


def kernel(x, edge_index, enc_w1, enc_b1, enc_w2, enc_b2,
           s1_wl, s1_bl, s1_wr, s2_wl, s2_bl, s2_wr,
           ro_w1, ro_b1, ro_w2, ro_b2):
    xp = jnp.pad(x, ((0, NPAD - N), (0, 0)))

    src = edge_index[0]
    dst = edge_index[1]
    src_p = jnp.concatenate(
        [src, jnp.zeros((EPAD - E,), dtype=jnp.int32)])
    dst_p = jnp.concatenate(
        [dst, jnp.full((EPAD - E,), N, dtype=jnp.int32)])
    idx8 = (src_p[None, :] * NCH
            + jnp.arange(NCH, dtype=jnp.int32)[:, None]).reshape(
                NCH, EPAD // BB, BB)
    dstr = dst_p.reshape(EPAD // BB, BB)
    zeros_h = jnp.zeros((STRIPE, CW), dtype=jnp.float32)
    ones_h = jnp.ones((BB, CW), dtype=jnp.float32)

    h = _encoder(xp, enc_w1, enc_b1, enc_w2, enc_b2)

    agg1, cnt8 = _make_sc_kernel(True)(h.reshape(NPAD * NCH, CW), idx8, dstr,
                                       zeros_h, ones_h)
    cnt = cnt8[:, :1]

    h1 = _combine(agg1, cnt, h, s1_wl, s1_bl, s1_wr)

    agg2 = _make_sc_kernel(False)(h1.reshape(NPAD * NCH, CW), idx8, dstr,
                                  zeros_h, ones_h)

    out = _combine_readout(agg2, cnt, h1, s2_wl, s2_bl, s2_wr,
                           ro_w1, ro_b1, ro_w2, ro_b2)
    return out[:N]


# speedup vs baseline: 1.2641x; 1.2641x over previous
import functools

import jax
import jax.numpy as jnp
from jax import lax
from jax.experimental import pallas as pl
from jax.experimental.pallas import tpu as pltpu
from jax.experimental.pallas import tpu_sc as plsc

N = 50000
NPAD = 50176
IN_DIM = 24
HID = 256
OUT_DIM = 12
E = 800000
EPAD = 819200
NC = 2
NS = 16
CW = 32
NCH = HID // CW
EPT = EPAD // NS
BB = 256
KK = 20
NR = 2
NSUPER = EPT // (BB * KK)
STRIPE = NPAD // NS


def _dot(a, b):
    return lax.dot_general(a, b, (((1,), (0,)), ((), ())),
                           preferred_element_type=jnp.float32)


_BR = 1024
_GRID = NPAD // _BR


def _enc_body(x_ref, w1_ref, b1_ref, w2_ref, b2_ref, o_ref):
    h = jnp.maximum(_dot(x_ref[...], w1_ref[...]) + b1_ref[...], 0.0)
    o_ref[...] = jnp.maximum(_dot(h, w2_ref[...]) + b2_ref[...], 0.0)


def _encoder(xp, w1, b1, w2, b2):
    return pl.pallas_call(
        _enc_body,
        grid=(_GRID,),
        in_specs=[
            pl.BlockSpec((_BR, IN_DIM), lambda i: (i, 0)),
            pl.BlockSpec((IN_DIM, HID // 2), lambda i: (0, 0)),
            pl.BlockSpec((1, HID // 2), lambda i: (0, 0)),
            pl.BlockSpec((HID // 2, HID), lambda i: (0, 0)),
            pl.BlockSpec((1, HID), lambda i: (0, 0)),
        ],
        out_specs=pl.BlockSpec((_BR, HID), lambda i: (i, 0)),
        out_shape=jax.ShapeDtypeStruct((NPAD, HID), jnp.float32),
    )(xp, w1, b1.reshape(1, -1), w2, b2.reshape(1, -1))


def _combine_body(agg_ref, cnt_ref, h_ref, wl_ref, bl_ref, wr_ref, o_ref):
    inv = 1.0 / jnp.maximum(cnt_ref[...], 1.0)
    mean = agg_ref[...] * inv
    o_ref[...] = jnp.maximum(
        _dot(mean, wl_ref[...]) + bl_ref[...] + _dot(h_ref[...], wr_ref[...]),
        0.0)


def _combine(agg, cnt, h, wl, bl, wr):
    return pl.pallas_call(
        _combine_body,
        grid=(_GRID,),
        in_specs=[
            pl.BlockSpec((_BR, HID), lambda i: (i, 0)),
            pl.BlockSpec((_BR, 1), lambda i: (i, 0)),
            pl.BlockSpec((_BR, HID), lambda i: (i, 0)),
            pl.BlockSpec((HID, HID), lambda i: (0, 0)),
            pl.BlockSpec((1, HID), lambda i: (0, 0)),
            pl.BlockSpec((HID, HID), lambda i: (0, 0)),
        ],
        out_specs=pl.BlockSpec((_BR, HID), lambda i: (i, 0)),
        out_shape=jax.ShapeDtypeStruct((NPAD, HID), jnp.float32),
    )(agg, cnt, h, wl, bl.reshape(1, -1), wr)


def _combine_ro_body(agg_ref, cnt_ref, h_ref, wl_ref, bl_ref, wr_ref,
                     rw1_ref, rb1_ref, rw2_ref, rb2_ref, o_ref):
    inv = 1.0 / jnp.maximum(cnt_ref[...], 1.0)
    mean = agg_ref[...] * inv
    t = jnp.maximum(
        _dot(mean, wl_ref[...]) + bl_ref[...] + _dot(h_ref[...], wr_ref[...]),
        0.0)
    t = jnp.maximum(_dot(t, rw1_ref[...]) + rb1_ref[...], 0.0)
    o_ref[...] = _dot(t, rw2_ref[...]) + rb2_ref[...]


def _combine_readout(agg, cnt, h, wl, bl, wr, rw1, rb1, rw2, rb2):
    return pl.pallas_call(
        _combine_ro_body,
        grid=(_GRID,),
        in_specs=[
            pl.BlockSpec((_BR, HID), lambda i: (i, 0)),
            pl.BlockSpec((_BR, 1), lambda i: (i, 0)),
            pl.BlockSpec((_BR, HID), lambda i: (i, 0)),
            pl.BlockSpec((HID, HID), lambda i: (0, 0)),
            pl.BlockSpec((1, HID), lambda i: (0, 0)),
            pl.BlockSpec((HID, HID), lambda i: (0, 0)),
            pl.BlockSpec((HID, HID // 2), lambda i: (0, 0)),
            pl.BlockSpec((1, HID // 2), lambda i: (0, 0)),
            pl.BlockSpec((HID // 2, OUT_DIM), lambda i: (0, 0)),
            pl.BlockSpec((1, OUT_DIM), lambda i: (0, 0)),
        ],
        out_specs=pl.BlockSpec((_BR, OUT_DIM), lambda i: (i, 0)),
        out_shape=jax.ShapeDtypeStruct((NPAD, OUT_DIM), jnp.float32),
    )(agg, cnt, h, wl, bl.reshape(1, -1), wr,
      rw1, rb1.reshape(1, -1), rw2, rb2.reshape(1, -1))


@functools.cache
def _make_sc_kernel(with_count):
    mesh = plsc.VectorSubcoreMesh(core_axis_name="c", subcore_axis_name="s",
                                  num_cores=NC, num_subcores=NS)
    out_type = [jax.ShapeDtypeStruct((NPAD, HID), jnp.float32)]
    if with_count:
        out_type.append(jax.ShapeDtypeStruct((NPAD, CW), jnp.float32))

    scratch = (
        [pltpu.VMEM((KK, BB), jnp.int32)]
        + [pltpu.VMEM((KK, BB), jnp.int32)]
        + [pltpu.VMEM((BB, CW), jnp.float32) for _ in range(NR)]
        + [pltpu.VMEM_SHARED((NPAD, CW), jnp.float32)]
        + [pltpu.SemaphoreType.DMA for _ in range(2 * NR)])

    def body(h8, idx8, dstr, zeros_h, ones_h, *rest):
        if with_count:
            agg_out, cnt_out = rest[0], rest[1]
            rest = rest[2:]
        else:
            agg_out = rest[0]
            cnt_out = None
            rest = rest[1:]
        idx_all, dst_all = rest[0], rest[1]
        rows = rest[2:2 + NR]
        slab = rest[2 + NR]
        gsem = rest[3 + NR:3 + 2 * NR]
        ssem = rest[3 + 2 * NR:3 + 3 * NR]

        cid = lax.axis_index("c")
        sid = lax.axis_index("s")
        stripe0 = sid * STRIPE

        def zero_slab():
            pltpu.sync_copy(zeros_h, slab.at[pl.ds(stripe0, STRIPE), :])

        def gather_pass(ch):
            zero_slab()
            plsc.subcore_barrier()

            def superb(sb, carry):
                row0 = sid * (EPT // BB) + sb * KK
                pltpu.sync_copy(dstr.at[pl.ds(row0, KK), :], dst_all)
                pltpu.sync_copy(idx8.at[ch, pl.ds(row0, KK), :], idx_all)
                gd = [None] * NR
                sd = [None] * NR
                for k in range(KK):
                    buf = k % NR
                    if sd[buf] is not None:
                        sd[buf].wait()
                    gd[buf] = pltpu.async_copy(
                        h8.at[idx_all.at[k]], rows[buf], gsem[buf])
                    if k >= NR - 1:
                        j = k - (NR - 1)
                        jb = j % NR
                        gd[jb].wait()
                        sd[jb] = pltpu.async_copy(
                            rows[jb], slab.at[dst_all.at[j]],
                            ssem[jb], add=True)
                for j in range(KK - (NR - 1), KK):
                    jb = j % NR
                    gd[jb].wait()
                    sd[jb] = pltpu.async_copy(
                        rows[jb], slab.at[dst_all.at[j]],
                        ssem[jb], add=True)
                for d in sd:
                    if d is not None:
                        d.wait()
                return carry

            lax.fori_loop(0, NSUPER, superb, 0)
            plsc.subcore_barrier()
            pltpu.sync_copy(
                slab.at[pl.ds(stripe0, STRIPE), :],
                agg_out.at[pl.ds(stripe0, STRIPE), pl.ds(ch * CW, CW)])
            plsc.subcore_barrier()

        for j in range(NCH // NC):
            gather_pass(cid * (NCH // NC) + j)

        if with_count:
            @pl.when(cid == 0)
            def _count_pass():
                zero_slab()
                pltpu.sync_copy(ones_h, rows[0])
                plsc.subcore_barrier()

                def csuperb(sb, carry):
                    row0 = sid * (EPT // BB) + sb * KK
                    pltpu.sync_copy(dstr.at[pl.ds(row0, KK), :], dst_all)
                    sds = []
                    for k in range(KK):
                        sds.append(pltpu.async_copy(
                            rows[0], slab.at[dst_all.at[k]], ssem[0],
                            add=True))
                    for sd in sds:
                        sd.wait()
                    return carry

                lax.fori_loop(0, NSUPER, csuperb, 0)
                plsc.subcore_barrier()
                pltpu.sync_copy(slab.at[pl.ds(stripe0, STRIPE), :],
                                cnt_out.at[pl.ds(stripe0, STRIPE), :])

    return pl.kernel(
        body,
        out_type=tuple(out_type) if with_count else out_type[0],
        mesh=mesh,
        scratch_types=scratch,
        compiler_params=pltpu.CompilerParams(use_tc_tiling_on_sc=False),
    )


def kernel(x, edge_index, enc_w1, enc_b1, enc_w2, enc_b2,
           s1_wl, s1_bl, s1_wr, s2_wl, s2_bl, s2_wr,
           ro_w1, ro_b1, ro_w2, ro_b2):
    xp = jnp.pad(x, ((0, NPAD - N), (0, 0)))

    src = edge_index[0]
    dst = edge_index[1]
    src_p = jnp.concatenate(
        [src, jnp.zeros((EPAD - E,), dtype=jnp.int32)])
    dst_p = jnp.concatenate(
        [dst, jnp.full((EPAD - E,), N, dtype=jnp.int32)])
    idx8 = (src_p[None, :] * NCH
            + jnp.arange(NCH, dtype=jnp.int32)[:, None]).reshape(
                NCH, EPAD // BB, BB)
    dstr = dst_p.reshape(EPAD // BB, BB)
    zeros_h = jnp.zeros((STRIPE, CW), dtype=jnp.float32)
    ones_h = jnp.ones((BB, CW), dtype=jnp.float32)

    h = _encoder(xp, enc_w1, enc_b1, enc_w2, enc_b2)

    agg1, cnt8 = _make_sc_kernel(True)(h.reshape(NPAD * NCH, CW), idx8, dstr,
                                       zeros_h, ones_h)
    cnt = cnt8[:, :1]

    h1 = _combine(agg1, cnt, h, s1_wl, s1_bl, s1_wr)

    agg2 = _make_sc_kernel(False)(h1.reshape(NPAD * NCH, CW), idx8, dstr,
                                  zeros_h, ones_h)

    out = _combine_readout(agg2, cnt, h1, s2_wl, s2_bl, s2_wr,
                           ro_w1, ro_b1, ro_w2, ro_b2)
    return out[:N]
